# baseline (speedup 1.0000x reference)
import functools

import jax
import jax.numpy as jnp
from jax import lax
from jax.experimental import pallas as pl
from jax.experimental.pallas import tpu as pltpu
from jax.experimental.pallas import tpu_sc as plsc

ROWS, COLS = 128, 32768
LANES = 16
SEG = COLS // LANES
NUM_CORES = 2
NUM_SUBCORES = 16
NUM_WORKERS = NUM_CORES * NUM_SUBCORES
ROWS_PER_WORKER = ROWS // NUM_WORKERS

_mesh = plsc.VectorSubcoreMesh(core_axis_name="c", subcore_axis_name="s")


@functools.partial(
    pl.kernel,
    out_type=jax.ShapeDtypeStruct((ROWS, COLS), jnp.float32),
    mesh=_mesh,
    scratch_types=[pltpu.VMEM((COLS,), jnp.float32)],
    compiler_params=pltpu.CompilerParams(needs_layout_passes=False),
)
def _scan_rows(x_hbm, out_hbm, buf):
    wid = lax.axis_index("s") * NUM_CORES + lax.axis_index("c")
    base_idx = lax.iota(jnp.int32, LANES) * SEG
    for r in range(ROWS_PER_WORKER):
        row = wid * ROWS_PER_WORKER + r
        pltpu.sync_copy(x_hbm.at[row], buf)

        def _sum_body(k, acc):
            return acc + plsc.load_gather(buf, [base_idx + k])

        seg_sums = lax.fori_loop(
            0, SEG, _sum_body, jnp.zeros((LANES,), jnp.float32)
        )
        lane_base = plsc.cumsum(seg_sums) - seg_sums

        def _scan_body(k, carry):
            idx = base_idx + k
            v = plsc.load_gather(buf, [idx])
            plsc.store_scatter(buf, [idx], carry)
            return carry + v

        lax.fori_loop(0, SEG, _scan_body, lane_base)
        pltpu.sync_copy(buf, out_hbm.at[row])


def kernel(x):
    return _scan_rows(x)


# speedup vs baseline: 1.5234x; 1.5234x over previous
import functools

import jax
import jax.numpy as jnp
from jax import lax
from jax.experimental import pallas as pl
from jax.experimental.pallas import tpu as pltpu
from jax.experimental.pallas import tpu_sc as plsc

ROWS, COLS = 128, 32768
L = 16
NUM_CORES = 2
NUM_WORKERS = 32
RPW = ROWS // NUM_WORKERS
CHUNK = 16384
CPR = COLS // CHUNK
SEGC = CHUNK // L
NT = RPW * CPR

_mesh = plsc.VectorSubcoreMesh(core_axis_name="c", subcore_axis_name="s")


@functools.partial(
    pl.kernel,
    out_type=jax.ShapeDtypeStruct((ROWS * COLS,), jnp.float32),
    mesh=_mesh,
    scratch_types=[
        pltpu.VMEM((CHUNK,), jnp.float32),
        pltpu.VMEM((CHUNK,), jnp.float32),
        pltpu.VMEM((CHUNK,), jnp.float32),
        pltpu.VMEM((CHUNK,), jnp.float32),
        pltpu.SemaphoreType.DMA,
        pltpu.SemaphoreType.DMA,
        pltpu.SemaphoreType.DMA,
        pltpu.SemaphoreType.DMA,
    ],
    compiler_params=pltpu.CompilerParams(needs_layout_passes=False),
)
def _scan_rows(x_hbm, out_hbm, in0, in1, out0, out1, si0, si1, so0, so1):
    wid = lax.axis_index("s") * NUM_CORES + lax.axis_index("c")
    inb, outb = (in0, in1), (out0, out1)
    sin, sout = (si0, si1), (so0, so1)
    base_idx = lax.iota(jnp.int32, L) * SEGC

    def hbm_off(t):
        row = wid * RPW + t // CPR
        return row * COLS + (t % CPR) * CHUNK

    loads = [None] * NT
    stores = [None] * NT
    loads[0] = pltpu.async_copy(x_hbm.at[pl.ds(hbm_off(0), CHUNK)], inb[0], sin[0])

    row_carry = jnp.float32(0)
    for t in range(NT):
        s = t % 2
        loads[t].wait()
        if t + 1 < NT:
            loads[t + 1] = pltpu.async_copy(
                x_hbm.at[pl.ds(hbm_off(t + 1), CHUNK)], inb[1 - s], sin[1 - s]
            )
        if t % CPR == 0:
            row_carry = jnp.float32(0)

        ib, ob = inb[s], outb[s]
        z = jnp.zeros((L,), jnp.float32)

        @plsc.parallel_loop(0, SEGC, step=4, carry=(z, z, z, z))
        def _pass_a(k, accs):
            a0, a1, a2, a3 = accs
            a0 = a0 + plsc.load_gather(ib, [base_idx + k])
            a1 = a1 + plsc.load_gather(ib, [base_idx + (k + 1)])
            a2 = a2 + plsc.load_gather(ib, [base_idx + (k + 2)])
            a3 = a3 + plsc.load_gather(ib, [base_idx + (k + 3)])
            return a0, a1, a2, a3

        a0, a1, a2, a3 = _pass_a
        seg_sums = (a0 + a1) + (a2 + a3)
        inc = plsc.cumsum(seg_sums)
        lane_base = (inc - seg_sums) + row_carry
        row_carry = row_carry + jnp.sum(seg_sums)

        if t >= 2:
            stores[t - 2].wait()

        @plsc.parallel_loop(0, SEGC, step=4, carry=lane_base)
        def _pass_b(k, carry):
            v0 = plsc.load_gather(ib, [base_idx + k])
            v1 = plsc.load_gather(ib, [base_idx + (k + 1)])
            v2 = plsc.load_gather(ib, [base_idx + (k + 2)])
            v3 = plsc.load_gather(ib, [base_idx + (k + 3)])
            plsc.store_scatter(ob, [base_idx + k], carry)
            c1 = carry + v0
            plsc.store_scatter(ob, [base_idx + (k + 1)], c1)
            c2 = c1 + v1
            plsc.store_scatter(ob, [base_idx + (k + 2)], c2)
            c3 = c2 + v2
            plsc.store_scatter(ob, [base_idx + (k + 3)], c3)
            return c3 + v3

        del _pass_b
        stores[t] = pltpu.async_copy(
            ob, out_hbm.at[pl.ds(hbm_off(t), CHUNK)], sout[s]
        )

    stores[NT - 2].wait()
    stores[NT - 1].wait()


def kernel(x):
    return _scan_rows(x.reshape(-1)).reshape(ROWS, COLS)
